# baseline (speedup 1.0000x reference)
import functools
import math

import jax
import jax.numpy as jnp
from jax.experimental import pallas as pl
from jax.experimental.pallas import tpu as pltpu

N_NODES = 10000
E_EDGES = 320000
IN_DIM = 128
D = 128
N_ETYPE = 38
N_LAYER = 2
HEADS = 4
DPH = D // HEADS
NUM_TAGS = 9

_ROWS_BLK = 1000


def _gelu(x):
    return jax.nn.gelu(x, approximate=False)


def _mm_body(x_ref, w_ref, b_ref, o_ref, *, act):
    y = jnp.dot(x_ref[...], w_ref[...], preferred_element_type=jnp.float32)
    y = y + b_ref[...]
    if act == "relu":
        y = jnp.maximum(y, 0.0)
    o_ref[...] = y


def _mm(x, w, b, act="none"):
    n, k = x.shape
    m = w.shape[1]
    grid = (n // _ROWS_BLK,)
    return pl.pallas_call(
        functools.partial(_mm_body, act=act),
        grid=grid,
        in_specs=[
            pl.BlockSpec((_ROWS_BLK, k), lambda i: (i, 0)),
            pl.BlockSpec((k, m), lambda i: (0, 0)),
            pl.BlockSpec((1, m), lambda i: (0, 0)),
        ],
        out_specs=pl.BlockSpec((_ROWS_BLK, m), lambda i: (i, 0)),
        out_shape=jax.ShapeDtypeStruct((n, m), jnp.float32),
    )(x, w, b.reshape(1, m))


def _mm2_body(x_ref, y_ref, wx_ref, wy_ref, b_ref, o_ref):
    z = jnp.dot(x_ref[...], wx_ref[...], preferred_element_type=jnp.float32)
    z = z + jnp.dot(y_ref[...], wy_ref[...], preferred_element_type=jnp.float32)
    o_ref[...] = z + b_ref[...]


def _mm2(x, y, wx, wy, b):
    n, k = x.shape
    m = wx.shape[1]
    grid = (n // _ROWS_BLK,)
    return pl.pallas_call(
        _mm2_body,
        grid=grid,
        in_specs=[
            pl.BlockSpec((_ROWS_BLK, k), lambda i: (i, 0)),
            pl.BlockSpec((_ROWS_BLK, y.shape[1]), lambda i: (i, 0)),
            pl.BlockSpec((k, m), lambda i: (0, 0)),
            pl.BlockSpec((y.shape[1], m), lambda i: (0, 0)),
            pl.BlockSpec((1, m), lambda i: (0, 0)),
        ],
        out_specs=pl.BlockSpec((_ROWS_BLK, m), lambda i: (i, 0)),
        out_shape=jax.ShapeDtypeStruct((n, m), jnp.float32),
    )(x, y, wx, wy, b.reshape(1, m))


_T_STEPS = N_NODES


def _viterbi_body(em_ref, s0_ref, transP_ref, endP_ref, out_ref, hist_ref):
    lane_iota = jax.lax.broadcasted_iota(jnp.int32, (1, 128), 1)
    big = jnp.int32(127)
    trows = [transP_ref[i:i + 1, :] for i in range(NUM_TAGS)]

    def fwd(t, srow):
        em_row = em_ref[pl.ds(t, 1), :]
        vals = []
        for i in range(NUM_TAGS):
            spl = jnp.broadcast_to(srow[:, i:i + 1], (1, 128))
            vals.append((spl + trows[i]) + em_row)
        idxs = [jnp.full((1, 128), i, jnp.int32) for i in range(NUM_TAGS)]
        while len(vals) > 1:
            nv, ni = [], []
            for a in range(0, len(vals) - 1, 2):
                keep = vals[a] >= vals[a + 1]
                nv.append(jnp.where(keep, vals[a], vals[a + 1]))
                ni.append(jnp.where(keep, idxs[a], idxs[a + 1]))
            if len(vals) % 2:
                nv.append(vals[-1])
                ni.append(idxs[-1])
            vals, idxs = nv, ni
        hist_ref[pl.ds(t - 1, 1), :] = idxs[0]
        return vals[0]

    srow = jax.lax.fori_loop(1, _T_STEPS, fwd, s0_ref[...])

    final = srow + endP_ref[...]
    fm = jnp.max(final)
    last = jnp.min(jnp.where(final == fm, lane_iota, big))
    last_row = jnp.zeros((1, 128), jnp.int32) + last
    out_ref[pl.ds(_T_STEPS - 1, 1), :] = last_row

    def bwd(ii, tag_row):
        t = _T_STEPS - 1 - ii
        row = hist_ref[pl.ds(t - 1, 1), :]
        val = jnp.max(jnp.where(lane_iota == tag_row, row, -1))
        val_row = jnp.zeros((1, 128), jnp.int32) + val
        out_ref[pl.ds(t - 1, 1), :] = val_row
        return val_row

    jax.lax.fori_loop(0, _T_STEPS - 1, bwd, last_row)


def _viterbi_pallas(em_full, s0row, transP, endP):
    out_rows = pl.pallas_call(
        _viterbi_body,
        in_specs=[
            pl.BlockSpec(memory_space=pltpu.VMEM),
            pl.BlockSpec(memory_space=pltpu.VMEM),
            pl.BlockSpec(memory_space=pltpu.VMEM),
            pl.BlockSpec(memory_space=pltpu.VMEM),
        ],
        out_specs=pl.BlockSpec(memory_space=pltpu.VMEM),
        out_shape=jax.ShapeDtypeStruct((_T_STEPS, 128), jnp.int32),
        scratch_shapes=[pltpu.VMEM((_T_STEPS, 128), jnp.int32)],
    )(em_full, s0row, transP, endP)
    return out_rows[:, 0][None, :]


def _edge_phase(Qn, K1, M1, K2, M2, src, dst, et, deg):
    Et = src.shape[0]
    q = Qn[src].reshape(Et, HEADS, DPH)
    k = (K1[dst] + K2[et]).reshape(Et, HEADS, DPH)
    scores = (q * k).sum(axis=2) * (1.0 / math.sqrt(DPH))
    gmax = jnp.max(scores)
    e = jnp.exp(scores - gmax)
    dsum = jax.ops.segment_sum(e, src, num_segments=N_NODES)
    alpha = e / (dsum[src] + 1e-16) * deg[src][:, None]
    msg = (M1[src] + M2[et]).reshape(Et, HEADS, DPH)
    out_e = (msg * alpha[:, :, None]).reshape(Et, D)
    return jax.ops.segment_sum(out_e, dst, num_segments=N_NODES)


def kernel(node_emb, params, edge_index, edge_type):
    p = params
    N = N_NODES
    inv_bn = 1.0 / math.sqrt(1.0 + 1e-5)

    H = _gelu(_mm(node_emb, p['lm2gnn_w'], p['lm2gnn_b']))
    X = H

    src = jnp.concatenate([edge_index[0], jnp.arange(N, dtype=edge_index.dtype)])
    dst = jnp.concatenate([edge_index[1], jnp.arange(N, dtype=edge_index.dtype)])
    et = jnp.concatenate([edge_type, jnp.full((N,), N_ETYPE, edge_type.dtype)])
    Et = src.shape[0]
    deg = jax.ops.segment_sum(jnp.ones((Et,), jnp.float32), src, num_segments=N)

    for l in range(N_LAYER):
        h = p['edge_w1'][l] + p['edge_b1'][l]
        h = jnp.maximum(h * inv_bn * p['edge_g1'][l] + p['edge_be1'][l], 0.0)
        table = h @ p['edge_w2'][l] + p['edge_b2'][l]
        K2 = table @ p['k_w'][l][D:] + p['k_b'][l]
        M2 = table @ p['m_w'][l][D:] + p['m_b'][l]

        Wcat = jnp.concatenate([p['q_w'][l], p['k_w'][l][:D], p['m_w'][l][:D]], axis=1)
        bcat = jnp.concatenate([p['q_b'][l], jnp.zeros((2 * D,), jnp.float32)])
        QKM = _mm(X, Wcat, bcat)
        Qn, K1, M1 = QKM[:, :D], QKM[:, D:2 * D], QKM[:, 2 * D:]

        aggr = _edge_phase(Qn, K1, M1, K2, M2, src, dst, et, deg)

        g1 = _mm(aggr, p['mlp_w1'][l] * inv_bn * p['mlp_g'][l][None, :],
                 p['mlp_b1'][l] * inv_bn * p['mlp_g'][l] + p['mlp_be'][l], act="relu")
        X = _gelu(_mm(g1, p['mlp_w2'][l], p['mlp_b2'][l]))

    hidden = _gelu(_mm2(H, X, p['fo_w'], p['fc_w'], p['fo_b'] + p['fc_b']))
    tag_w = jnp.zeros((D, 128), jnp.float32).at[:, :NUM_TAGS].set(p['tag_w'])
    tag_b = jnp.zeros((128,), jnp.float32).at[:NUM_TAGS].set(p['tag_b'])
    em_full = _mm(hidden, tag_w, tag_b)

    start, end, trans = p['crf_start'], p['crf_end'], p['crf_trans']
    NEG = jnp.float32(-1e30)
    transP = jnp.full((16, 128), NEG).at[:NUM_TAGS, :NUM_TAGS].set(trans)
    endP = jnp.full((1, 128), NEG).at[0, :NUM_TAGS].set(end)
    s0row = jnp.full((1, 128), NEG).at[0, :NUM_TAGS].set(start + em_full[0, :NUM_TAGS])
    tags = _viterbi_pallas(em_full, s0row, transP, endP)
    return tags


# speedup vs baseline: 1.5047x; 1.5047x over previous
import functools
import math

import jax
import jax.numpy as jnp
from jax import lax
from jax.experimental import pallas as pl
from jax.experimental.pallas import tpu as pltpu
from jax.experimental.pallas import tpu_sc as plsc

N_NODES = 10000
E_EDGES = 320000
IN_DIM = 128
D = 128
N_ETYPE = 38
N_LAYER = 2
HEADS = 4
DPH = D // HEADS
NUM_TAGS = 9

_ROWS_BLK = 1000


def _gelu(x):
    return jax.nn.gelu(x, approximate=False)


def _mm_body(x_ref, w_ref, b_ref, o_ref, *, act):
    y = jnp.dot(x_ref[...], w_ref[...], preferred_element_type=jnp.float32)
    y = y + b_ref[...]
    if act == "relu":
        y = jnp.maximum(y, 0.0)
    o_ref[...] = y


def _mm(x, w, b, act="none"):
    n, k = x.shape
    m = w.shape[1]
    grid = (n // _ROWS_BLK,)
    return pl.pallas_call(
        functools.partial(_mm_body, act=act),
        grid=grid,
        in_specs=[
            pl.BlockSpec((_ROWS_BLK, k), lambda i: (i, 0)),
            pl.BlockSpec((k, m), lambda i: (0, 0)),
            pl.BlockSpec((1, m), lambda i: (0, 0)),
        ],
        out_specs=pl.BlockSpec((_ROWS_BLK, m), lambda i: (i, 0)),
        out_shape=jax.ShapeDtypeStruct((n, m), jnp.float32),
    )(x, w, b.reshape(1, m))


def _mm2_body(x_ref, y_ref, wx_ref, wy_ref, b_ref, o_ref):
    z = jnp.dot(x_ref[...], wx_ref[...], preferred_element_type=jnp.float32)
    z = z + jnp.dot(y_ref[...], wy_ref[...], preferred_element_type=jnp.float32)
    o_ref[...] = z + b_ref[...]


def _mm2(x, y, wx, wy, b):
    n, k = x.shape
    m = wx.shape[1]
    grid = (n // _ROWS_BLK,)
    return pl.pallas_call(
        _mm2_body,
        grid=grid,
        in_specs=[
            pl.BlockSpec((_ROWS_BLK, k), lambda i: (i, 0)),
            pl.BlockSpec((_ROWS_BLK, y.shape[1]), lambda i: (i, 0)),
            pl.BlockSpec((k, m), lambda i: (0, 0)),
            pl.BlockSpec((y.shape[1], m), lambda i: (0, 0)),
            pl.BlockSpec((1, m), lambda i: (0, 0)),
        ],
        out_specs=pl.BlockSpec((_ROWS_BLK, m), lambda i: (i, 0)),
        out_shape=jax.ShapeDtypeStruct((n, m), jnp.float32),
    )(x, y, wx, wy, b.reshape(1, m))


_T_STEPS = N_NODES


def _viterbi_body(em_ref, s0_ref, transP_ref, endP_ref, out_ref, hist_ref):
    lane_iota = jax.lax.broadcasted_iota(jnp.int32, (1, 128), 1)
    big = jnp.int32(127)
    trows = [transP_ref[i:i + 1, :] for i in range(NUM_TAGS)]

    def fwd(t, srow):
        em_row = em_ref[pl.ds(t, 1), :]
        vals = []
        for i in range(NUM_TAGS):
            spl = jnp.broadcast_to(srow[:, i:i + 1], (1, 128))
            vals.append((spl + trows[i]) + em_row)
        idxs = [jnp.full((1, 128), i, jnp.int32) for i in range(NUM_TAGS)]
        while len(vals) > 1:
            nv, ni = [], []
            for a in range(0, len(vals) - 1, 2):
                keep = vals[a] >= vals[a + 1]
                nv.append(jnp.where(keep, vals[a], vals[a + 1]))
                ni.append(jnp.where(keep, idxs[a], idxs[a + 1]))
            if len(vals) % 2:
                nv.append(vals[-1])
                ni.append(idxs[-1])
            vals, idxs = nv, ni
        hist_ref[pl.ds(t - 1, 1), :] = idxs[0]
        return vals[0]

    srow = jax.lax.fori_loop(1, _T_STEPS, fwd, s0_ref[...])

    final = srow + endP_ref[...]
    fm = jnp.max(final)
    last = jnp.min(jnp.where(final == fm, lane_iota, big))
    last_row = jnp.zeros((1, 128), jnp.int32) + last
    out_ref[pl.ds(_T_STEPS - 1, 1), :] = last_row

    def bwd(ii, tag_row):
        t = _T_STEPS - 1 - ii
        row = hist_ref[pl.ds(t - 1, 1), :]
        val = jnp.max(jnp.where(lane_iota == tag_row, row, -1))
        val_row = jnp.zeros((1, 128), jnp.int32) + val
        out_ref[pl.ds(t - 1, 1), :] = val_row
        return val_row

    jax.lax.fori_loop(0, _T_STEPS - 1, bwd, last_row)


def _viterbi_pallas(em_full, s0row, transP, endP):
    out_rows = pl.pallas_call(
        _viterbi_body,
        in_specs=[
            pl.BlockSpec(memory_space=pltpu.VMEM),
            pl.BlockSpec(memory_space=pltpu.VMEM),
            pl.BlockSpec(memory_space=pltpu.VMEM),
            pl.BlockSpec(memory_space=pltpu.VMEM),
        ],
        out_specs=pl.BlockSpec(memory_space=pltpu.VMEM),
        out_shape=jax.ShapeDtypeStruct((_T_STEPS, 128), jnp.int32),
        scratch_shapes=[pltpu.VMEM((_T_STEPS, 128), jnp.int32)],
    )(em_full, s0row, transP, endP)
    return out_rows[:, 0][None, :]


_NW = 32
_CH = 120
_B_PAD = 330240


def _sc_gather(table, idx_pad, dcols):
    b_per_w = _B_PAD // _NW
    n_ch = b_per_w // _CH
    mesh = plsc.VectorSubcoreMesh(core_axis_name="c", subcore_axis_name="s")

    def body(table_hbm, idx_hbm, out_hbm, idx_v, rows_v, sem):
        wid = lax.axis_index("s") * 2 + lax.axis_index("c")
        base = wid * b_per_w

        def chunk(i, carry):
            off = base + i * _CH
            pltpu.sync_copy(idx_hbm.at[pl.ds(off, _CH)], idx_v)
            pltpu.async_copy(table_hbm.at[idx_v], rows_v, sem).wait()
            pltpu.sync_copy(rows_v, out_hbm.at[pl.ds(off, _CH)])
            return carry

        lax.fori_loop(0, n_ch, chunk, 0)

    f = pl.kernel(
        body,
        mesh=mesh,
        out_type=jax.ShapeDtypeStruct((_B_PAD, dcols), jnp.float32),
        scratch_types=[
            pltpu.VMEM((_CH,), jnp.int32),
            pltpu.VMEM((_CH, dcols), jnp.float32),
            pltpu.SemaphoreType.DMA,
        ],
    )
    return f(table, idx_pad)


def _edge_phase(Qn, K1, M1, K2, M2, src_pad, dst_pad, et_pad, deg, valid):
    B = _B_PAD
    QM = _sc_gather(jnp.concatenate([Qn, M1], axis=1), src_pad, 2 * D)
    q_s, m1_s = QM[:, :D], QM[:, D:]
    k1_d = _sc_gather(K1, dst_pad, D)
    m2_e = _sc_gather(M2, et_pad, D)
    score1 = (q_s.reshape(B, HEADS, DPH) * k1_d.reshape(B, HEADS, DPH)).sum(axis=2)
    T2 = jnp.einsum('nhd,thd->nth', Qn.reshape(N_NODES, HEADS, DPH),
                    K2.reshape(N_ETYPE + 1, HEADS, DPH))
    score2 = T2.reshape(N_NODES * (N_ETYPE + 1), HEADS)[src_pad * (N_ETYPE + 1) + et_pad]
    scores = (score1 + score2) * (1.0 / math.sqrt(DPH))
    gmax = jnp.max(jnp.where(valid, scores, -1e30))
    e = jnp.where(valid, jnp.exp(scores - gmax), 0.0)
    dsum = jax.ops.segment_sum(e, src_pad, num_segments=N_NODES)
    alpha = e / (dsum[src_pad] + 1e-16) * deg[src_pad][:, None]
    msg = m1_s + m2_e
    out_e = (msg.reshape(B, HEADS, DPH) * alpha[:, :, None]).reshape(B, D)
    return jax.ops.segment_sum(out_e, dst_pad, num_segments=N_NODES)


def kernel(node_emb, params, edge_index, edge_type):
    p = params
    N = N_NODES
    inv_bn = 1.0 / math.sqrt(1.0 + 1e-5)

    H = _gelu(_mm(node_emb, p['lm2gnn_w'], p['lm2gnn_b']))
    X = H

    src = jnp.concatenate([edge_index[0], jnp.arange(N, dtype=edge_index.dtype)])
    dst = jnp.concatenate([edge_index[1], jnp.arange(N, dtype=edge_index.dtype)])
    et = jnp.concatenate([edge_type, jnp.full((N,), N_ETYPE, edge_type.dtype)])
    Et = src.shape[0]
    deg = jax.ops.segment_sum(jnp.ones((Et,), jnp.float32), src, num_segments=N)
    npad = _B_PAD - Et
    zpad = jnp.zeros((npad,), src.dtype)
    src_pad = jnp.concatenate([src, zpad])
    dst_pad = jnp.concatenate([dst, zpad])
    et_pad = jnp.concatenate([et, zpad])
    valid = (jax.lax.broadcasted_iota(jnp.int32, (_B_PAD, 1), 0) < Et)

    for l in range(N_LAYER):
        h = p['edge_w1'][l] + p['edge_b1'][l]
        h = jnp.maximum(h * inv_bn * p['edge_g1'][l] + p['edge_be1'][l], 0.0)
        table = h @ p['edge_w2'][l] + p['edge_b2'][l]
        K2 = table @ p['k_w'][l][D:] + p['k_b'][l]
        M2 = table @ p['m_w'][l][D:] + p['m_b'][l]

        Wcat = jnp.concatenate([p['q_w'][l], p['k_w'][l][:D], p['m_w'][l][:D]], axis=1)
        bcat = jnp.concatenate([p['q_b'][l], jnp.zeros((2 * D,), jnp.float32)])
        QKM = _mm(X, Wcat, bcat)
        Qn, K1, M1 = QKM[:, :D], QKM[:, D:2 * D], QKM[:, 2 * D:]

        aggr = _edge_phase(Qn, K1, M1, K2, M2, src_pad, dst_pad, et_pad, deg, valid)

        g1 = _mm(aggr, p['mlp_w1'][l] * inv_bn * p['mlp_g'][l][None, :],
                 p['mlp_b1'][l] * inv_bn * p['mlp_g'][l] + p['mlp_be'][l], act="relu")
        X = _gelu(_mm(g1, p['mlp_w2'][l], p['mlp_b2'][l]))

    hidden = _gelu(_mm2(H, X, p['fo_w'], p['fc_w'], p['fo_b'] + p['fc_b']))
    tag_w = jnp.zeros((D, 128), jnp.float32).at[:, :NUM_TAGS].set(p['tag_w'])
    tag_b = jnp.zeros((128,), jnp.float32).at[:NUM_TAGS].set(p['tag_b'])
    em_full = _mm(hidden, tag_w, tag_b)

    start, end, trans = p['crf_start'], p['crf_end'], p['crf_trans']
    NEG = jnp.float32(-1e30)
    transP = jnp.full((16, 128), NEG).at[:NUM_TAGS, :NUM_TAGS].set(trans)
    endP = jnp.full((1, 128), NEG).at[0, :NUM_TAGS].set(end)
    s0row = jnp.full((1, 128), NEG).at[0, :NUM_TAGS].set(start + em_full[0, :NUM_TAGS])
    tags = _viterbi_pallas(em_full, s0row, transP, endP)
    return tags
